# baseline (speedup 1.0000x reference)
import jax
import jax.numpy as jnp
from jax.experimental import pallas as pl

_N = 4096
_BR = 512
_BC = 1024


def _interior(r, c):
    return (r >= 1024 // _BR) & (r < 3072 // _BR) & (c >= 1) & (c <= 2)


def _body(b_ref, bt_ref, o_ref):
    r = pl.program_id(0)
    c = pl.program_id(1)
    inside = _interior(r, c)

    @pl.when(inside)
    def _fill():
        o_ref[...] = jnp.full((1, _BR, _BC), bt_ref[0, 0], jnp.float32)

    @pl.when(jnp.logical_not(inside))
    def _copy():
        o_ref[...] = b_ref[...]


def kernel(B, Bt):
    return pl.pallas_call(
        _body,
        grid=(_N // _BR, _N // _BC),
        in_specs=[
            pl.BlockSpec(
                (1, _BR, _BC),
                lambda r, c: (0, r, jnp.where(_interior(r, c), 0, c)),
            ),
            pl.BlockSpec((1, 1), lambda r, c: (0, 0)),
        ],
        out_specs=pl.BlockSpec((1, _BR, _BC), lambda r, c: (0, r, c)),
        out_shape=jax.ShapeDtypeStruct((1, _N, _N), jnp.float32),
    )(B, Bt)


# speedup vs baseline: 1.0772x; 1.0772x over previous
import jax
import jax.numpy as jnp
from jax.experimental import pallas as pl

_N = 4096
_BR = 512


def _body(b_ref, bt_ref, o_ref):
    i = pl.program_id(0)
    in_rows = (i >= 1024 // _BR) & (i < 3072 // _BR)

    @pl.when(in_rows)
    def _select():
        cols = jax.lax.broadcasted_iota(jnp.int32, (1, _BR, _N), 2)
        mask = (cols >= 1024) & (cols < 3072)
        o_ref[...] = jnp.where(mask, bt_ref[0, 0], b_ref[...])

    @pl.when(jnp.logical_not(in_rows))
    def _copy():
        o_ref[...] = b_ref[...]


def kernel(B, Bt):
    return pl.pallas_call(
        _body,
        grid=(_N // _BR,),
        in_specs=[
            pl.BlockSpec((1, _BR, _N), lambda i: (0, i, 0)),
            pl.BlockSpec((1, 1), lambda i: (0, 0)),
        ],
        out_specs=pl.BlockSpec((1, _BR, _N), lambda i: (0, i, 0)),
        out_shape=jax.ShapeDtypeStruct((1, _N, _N), jnp.float32),
    )(B, Bt)
